# Initial kernel scaffold; baseline (speedup 1.0000x reference)
import jax
import jax.numpy as jnp
from jax.experimental import pallas as pl


def kernel(x, edge_index, W_init, a_left, a_right, Ws, W_final, b_final):
    raise NotImplementedError("write your pallas kernel here")


# speedup vs baseline: 26.1440x; 26.1440x over previous
import functools

import jax
import jax.numpy as jnp
from jax import lax
from jax.experimental import pallas as pl
from jax.experimental.pallas import tpu as pltpu
from jax.experimental.pallas import tpu_sc as plsc

NN = 50000
NE = 800000
NH = 10
FH = 5
NL = 3
NC = 6
HF = NH * FH

NT = 16
EPT = NE // NT
CH = 80
SUB = 10
NSUB = CH // SUB
NCHUNK = EPT // CH
AW = 32
SW = 16
NP = 51200
RPT = NP // NT
ZR = 128

BB = 5000
NBLK = NN // BB


def _h_from_acc(acc, rep):
    hs = []
    for c in (0, 1):
        msg = acc[c, :, 0:25]
        w = acc[c, :, 25:30]
        den = jnp.dot(w, rep, preferred_element_type=jnp.float32) + 1e-9
        hs.append(jnp.maximum(msg / den, 0.0))
    return jnp.concatenate(hs, axis=1)


def _stage_a_emit(h2, wbd, alp, arp, bt_ref, src_ref, gmax_ref, first):
    hp = jnp.dot(h2, wbd, preferred_element_type=jnp.float32)
    src = jnp.dot(hp, alp, preferred_element_type=jnp.float32)
    nbr = jnp.dot(hp, arp, preferred_element_type=jnp.float32)
    pad = jnp.zeros((h2.shape[0], 2), jnp.float32)
    rows = [jnp.concatenate(
        [nbr[:, c * 5:(c + 1) * 5], hp[:, c * 25:(c + 1) * 25], pad], axis=1)
        for c in (0, 1)]
    bt_ref[...] = jnp.stack(rows, axis=0)
    src_ref[...] = src
    bm = jnp.max(nbr, axis=0, keepdims=True)

    @pl.when(first)
    def _():
        gmax_ref[...] = jnp.full((1, 16), -1e30, jnp.float32)

    gmax_ref[...] = jnp.maximum(gmax_ref[...], bm)


def _tc_a0_body(x_ref, wini_ref, wbd_ref, alp_ref, arp_ref,
                bt_ref, src_ref, gmax_ref):
    h2 = jnp.dot(x_ref[...], wini_ref[...], preferred_element_type=jnp.float32)
    _stage_a_emit(h2, wbd_ref[...], alp_ref[...], arp_ref[...],
                  bt_ref, src_ref, gmax_ref, pl.program_id(0) == 0)


def _tc_ai_body(acc_ref, rep_ref, wbd_ref, alp_ref, arp_ref,
                bt_ref, src_ref, gmax_ref):
    h2 = _h_from_acc(acc_ref[...], rep_ref[...])
    _stage_a_emit(h2, wbd_ref[...], alp_ref[...], arp_ref[...],
                  bt_ref, src_ref, gmax_ref, pl.program_id(0) == 0)


def _tc_b_body(src_ref, gmax_ref, srct_ref):
    src = src_ref[...]
    g = gmax_ref[...]
    halves = []
    pad = jnp.zeros((src.shape[0], 6), jnp.float32)
    for c in (0, 1):
        s = src[:, c * 5:(c + 1) * 5]
        marg = s + g[:, c * 5:(c + 1) * 5]
        m = jnp.maximum(marg, 0.2 * marg)
        halves.append(jnp.concatenate([s, m, pad], axis=1))
    srct_ref[...] = jnp.stack(halves, axis=0)


def _tc_final_body(acc_ref, rep_ref, wf_ref, bf_ref, out_ref):
    h2 = _h_from_acc(acc_ref[...], rep_ref[...])
    out_ref[...] = (jnp.dot(h2, wf_ref[...], preferred_element_type=jnp.float32)
                    + bf_ref[...])


def _full(shape):
    return pl.BlockSpec(shape, lambda b: tuple(0 for _ in shape))


_A_OUTS = [
    jax.ShapeDtypeStruct((2, NN, AW), jnp.float32),
    jax.ShapeDtypeStruct((NN, 16), jnp.float32),
    jax.ShapeDtypeStruct((1, 16), jnp.float32),
]
_A_OUT_SPECS = [
    pl.BlockSpec((2, BB, AW), lambda b: (0, b, 0)),
    pl.BlockSpec((BB, 16), lambda b: (b, 0)),
    pl.BlockSpec((1, 16), lambda b: (0, 0)),
]


def _tc_stage_a0(x, wini, wbd, alp, arp):
    return pl.pallas_call(
        _tc_a0_body,
        grid=(NBLK,),
        in_specs=[pl.BlockSpec((BB, 128), lambda b: (b, 0)),
                  _full((128, HF)), _full((HF, HF)),
                  _full((HF, 16)), _full((HF, 16))],
        out_specs=_A_OUT_SPECS,
        out_shape=_A_OUTS,
    )(x, wini, wbd, alp, arp)


def _tc_stage_ai(acc, rep, wbd, alp, arp):
    return pl.pallas_call(
        _tc_ai_body,
        grid=(NBLK,),
        in_specs=[pl.BlockSpec((2, BB, AW), lambda b: (0, b, 0)),
                  _full((FH, 25)), _full((HF, HF)),
                  _full((HF, 16)), _full((HF, 16))],
        out_specs=_A_OUT_SPECS,
        out_shape=_A_OUTS,
    )(acc, rep, wbd, alp, arp)


def _tc_stage_b(src_raw, gmax):
    return pl.pallas_call(
        _tc_b_body,
        grid=(NBLK,),
        in_specs=[pl.BlockSpec((BB, 16), lambda b: (b, 0)),
                  _full((1, 16))],
        out_specs=pl.BlockSpec((2, BB, SW), lambda b: (0, b, 0)),
        out_shape=jax.ShapeDtypeStruct((2, NN, SW), jnp.float32),
    )(src_raw, gmax)


def _tc_final(acc, rep, wf, bf):
    return pl.pallas_call(
        _tc_final_body,
        grid=(NBLK,),
        in_specs=[pl.BlockSpec((2, BB, AW), lambda b: (0, b, 0)),
                  _full((FH, 25)), _full((HF, NC)), _full((1, NC))],
        out_specs=pl.BlockSpec((BB, NC), lambda b: (b, 0)),
        out_shape=jax.ShapeDtypeStruct((NN, NC), jnp.float32),
    )(acc, rep, wf, bf)


def _sc_edge_body(rows_hbm, cols_hbm, srct_hbm, bt_hbm, out_hbm,
                  ridx, cidx, srcb, bb, outb, zbuf, accum, sem):
    c = lax.axis_index("c")
    s = lax.axis_index("s")
    zeros16 = jnp.zeros((16,), jnp.float32)

    def _zb(i, carry):
        zbuf[i, pl.ds(0, 16)] = zeros16
        zbuf[i, pl.ds(16, 16)] = zeros16
        return carry
    lax.fori_loop(0, ZR, _zb, 0)

    base_row = s * RPT

    def _zc(i, carry):
        pltpu.sync_copy(zbuf, accum.at[pl.ds(base_row + i * ZR, ZR)])
        return carry
    lax.fori_loop(0, RPT // ZR, _zc, 0)

    def _zo(i, carry):
        e16 = lax.iota(jnp.int32, 16) + i * 16
        plsc.store_scatter(outb, [e16, jnp.full((16,), 30, jnp.int32)], zeros16)
        plsc.store_scatter(outb, [e16, jnp.full((16,), 31, jnp.int32)], zeros16)
        return carry
    lax.fori_loop(0, CH // 16, _zo, 0)

    plsc.subcore_barrier()

    tile_r0 = s * (EPT // SUB)

    def _chunk(k, carry):
        r0 = tile_r0 + k * NSUB
        pltpu.sync_copy(rows_hbm.at[pl.ds(r0, NSUB)], ridx)
        pltpu.sync_copy(cols_hbm.at[pl.ds(r0, NSUB)], cidx)
        descs = []
        for j in range(NSUB):
            descs.append(pltpu.async_copy(
                srct_hbm.at[c].at[ridx.at[j]],
                srcb.at[pl.ds(j * SUB, SUB)], sem))
            descs.append(pltpu.async_copy(
                bt_hbm.at[c].at[cidx.at[j]],
                bb.at[pl.ds(j * SUB, SUB)], sem))
        for d in descs:
            d.wait()

        def _grp(g, carry2):
            e16 = lax.iota(jnp.int32, 16) + g * 16
            for h in range(5):
                hv = jnp.full((16,), h, jnp.int32)
                sv = plsc.load_gather(srcb, [e16, hv])
                mv = plsc.load_gather(srcb, [e16, hv + 5])
                nv = plsc.load_gather(bb, [e16, hv])
                pre = sv + nv
                w = jnp.exp(jnp.maximum(pre, 0.2 * pre) - mv)
                plsc.store_scatter(
                    outb, [e16, jnp.full((16,), 25 + h, jnp.int32)], w)
                for f in range(5):
                    hp = plsc.load_gather(
                        bb, [e16, jnp.full((16,), 5 + h * 5 + f, jnp.int32)])
                    plsc.store_scatter(
                        outb, [e16, jnp.full((16,), h * 5 + f, jnp.int32)],
                        hp * w)
            return carry2
        lax.fori_loop(0, CH // 16, _grp, 0)

        for j in range(NSUB):
            pltpu.sync_copy(outb.at[pl.ds(j * SUB, SUB)],
                            accum.at[ridx.at[j]], add=True)
        return carry
    lax.fori_loop(0, NCHUNK, _chunk, 0)

    plsc.subcore_barrier()

    def _out(i, carry):
        rr = pl.ds(base_row + i * ZR, ZR)
        pltpu.sync_copy(accum.at[rr], out_hbm.at[c].at[rr])
        return carry
    lax.fori_loop(0, RPT // ZR, _out, 0)


def _sc_edge(rows2d, cols2d, srct, bt):
    mesh = plsc.VectorSubcoreMesh(core_axis_name="c", subcore_axis_name="s")
    f = pl.kernel(
        _sc_edge_body,
        out_type=jax.ShapeDtypeStruct((2, NP, AW), jnp.float32),
        mesh=mesh,
        compiler_params=pltpu.CompilerParams(needs_layout_passes=False,
                                             use_tc_tiling_on_sc=False),
        scratch_types=[
            pltpu.VMEM((NSUB, SUB), jnp.int32),
            pltpu.VMEM((NSUB, SUB), jnp.int32),
            pltpu.VMEM((CH, SW), jnp.float32),
            pltpu.VMEM((CH, AW), jnp.float32),
            pltpu.VMEM((CH, AW), jnp.float32),
            pltpu.VMEM((ZR, AW), jnp.float32),
            pltpu.VMEM_SHARED((NP, AW), jnp.float32),
            pltpu.SemaphoreType.DMA,
        ],
    )
    return f(rows2d, cols2d, srct, bt)[:, :NN, :]


def kernel(x, edge_index, W_init, a_left, a_right, Ws, W_final, b_final):
    f32 = jnp.float32
    rows2d = edge_index[0].reshape(NE // SUB, SUB)
    cols2d = edge_index[1].reshape(NE // SUB, SUB)

    eye = jnp.eye(NH, dtype=f32)
    wbd = jnp.einsum('lhfo,hk->lhfko', Ws, eye).reshape(NL, HF, HF)
    alp = jnp.einsum('lhf,hk->lhfk', a_left, eye).reshape(NL, HF, NH)
    arp = jnp.einsum('lhf,hk->lhfk', a_right, eye).reshape(NL, HF, NH)
    zpad = jnp.zeros((NL, HF, 16 - NH), f32)
    alp = jnp.concatenate([alp, zpad], axis=2)
    arp = jnp.concatenate([arp, zpad], axis=2)
    rep = jnp.kron(jnp.eye(FH, dtype=f32), jnp.ones((1, FH), f32))

    bt, src_raw, gmax = _tc_stage_a0(x, W_init, wbd[0], alp[0], arp[0])
    srct = _tc_stage_b(src_raw, gmax)
    acc = _sc_edge(rows2d, cols2d, srct, bt)
    for i in range(1, NL):
        bt, src_raw, gmax = _tc_stage_ai(acc, rep, wbd[i], alp[i], arp[i])
        srct = _tc_stage_b(src_raw, gmax)
        acc = _sc_edge(rows2d, cols2d, srct, bt)
    return _tc_final(acc, rep, W_final, b_final.reshape(1, NC))


# speedup vs baseline: 41.5957x; 1.5910x over previous
import functools

import jax
import jax.numpy as jnp
from jax import lax
from jax.experimental import pallas as pl
from jax.experimental.pallas import tpu as pltpu
from jax.experimental.pallas import tpu_sc as plsc

NN = 50000
NE = 800000
NH = 10
FH = 5
NL = 3
NC = 6
HF = NH * FH

NT = 16
EPT = NE // NT
SUB = 50
CS = 100
NSC = EPT // CS
NIC = EPT // 400
AW = 32
SW = 16
NP = 51200
RPT = NP // NT
ZR = 128

BB = 5000
NBLK = NN // BB


def _h_from_acc(acc, rep):
    hs = []
    for c in (0, 1):
        msg = acc[c, :, 0:25]
        w = acc[c, :, 25:30]
        den = jnp.dot(w, rep, preferred_element_type=jnp.float32) + 1e-9
        hs.append(jnp.maximum(msg / den, 0.0))
    return jnp.concatenate(hs, axis=1)


def _stage_a_emit(h2, wbd, alp, arp, bt_ref, src_ref, gmax_ref, first):
    hp = jnp.dot(h2, wbd, preferred_element_type=jnp.float32)
    src = jnp.dot(hp, alp, preferred_element_type=jnp.float32)
    nbr = jnp.dot(hp, arp, preferred_element_type=jnp.float32)
    pad = jnp.zeros((h2.shape[0], 2), jnp.float32)
    rows = [jnp.concatenate(
        [nbr[:, c * 5:(c + 1) * 5], hp[:, c * 25:(c + 1) * 25], pad], axis=1)
        for c in (0, 1)]
    bt_ref[...] = jnp.stack(rows, axis=0)
    src_ref[...] = src
    bm = jnp.max(nbr, axis=0, keepdims=True)

    @pl.when(first)
    def _():
        gmax_ref[...] = jnp.full((1, 16), -1e30, jnp.float32)

    gmax_ref[...] = jnp.maximum(gmax_ref[...], bm)


def _tc_a0_body(x_ref, wini_ref, wbd_ref, alp_ref, arp_ref,
                bt_ref, src_ref, gmax_ref):
    h2 = jnp.dot(x_ref[...], wini_ref[...], preferred_element_type=jnp.float32)
    _stage_a_emit(h2, wbd_ref[...], alp_ref[...], arp_ref[...],
                  bt_ref, src_ref, gmax_ref, pl.program_id(0) == 0)


def _tc_ai_body(acc_ref, rep_ref, wbd_ref, alp_ref, arp_ref,
                bt_ref, src_ref, gmax_ref):
    h2 = _h_from_acc(acc_ref[...], rep_ref[...])
    _stage_a_emit(h2, wbd_ref[...], alp_ref[...], arp_ref[...],
                  bt_ref, src_ref, gmax_ref, pl.program_id(0) == 0)


def _tc_b_body(src_ref, gmax_ref, srct_ref):
    src = src_ref[...]
    g = gmax_ref[...]
    halves = []
    pad = jnp.zeros((src.shape[0], 6), jnp.float32)
    for c in (0, 1):
        s = src[:, c * 5:(c + 1) * 5]
        marg = s + g[:, c * 5:(c + 1) * 5]
        m = jnp.maximum(marg, 0.2 * marg)
        halves.append(jnp.concatenate([s, m, pad], axis=1))
    srct_ref[...] = jnp.stack(halves, axis=0)


def _tc_final_body(acc_ref, rep_ref, wf_ref, bf_ref, out_ref):
    h2 = _h_from_acc(acc_ref[...], rep_ref[...])
    out_ref[...] = (jnp.dot(h2, wf_ref[...], preferred_element_type=jnp.float32)
                    + bf_ref[...])


def _full(shape):
    return pl.BlockSpec(shape, lambda b: tuple(0 for _ in shape))


_A_OUTS = [
    jax.ShapeDtypeStruct((2, NN, AW), jnp.float32),
    jax.ShapeDtypeStruct((NN, 16), jnp.float32),
    jax.ShapeDtypeStruct((1, 16), jnp.float32),
]
_A_OUT_SPECS = [
    pl.BlockSpec((2, BB, AW), lambda b: (0, b, 0)),
    pl.BlockSpec((BB, 16), lambda b: (b, 0)),
    pl.BlockSpec((1, 16), lambda b: (0, 0)),
]


def _tc_stage_a0(x, wini, wbd, alp, arp):
    return pl.pallas_call(
        _tc_a0_body,
        grid=(NBLK,),
        in_specs=[pl.BlockSpec((BB, 128), lambda b: (b, 0)),
                  _full((128, HF)), _full((HF, HF)),
                  _full((HF, 16)), _full((HF, 16))],
        out_specs=_A_OUT_SPECS,
        out_shape=_A_OUTS,
    )(x, wini, wbd, alp, arp)


def _tc_stage_ai(acc, rep, wbd, alp, arp):
    return pl.pallas_call(
        _tc_ai_body,
        grid=(NBLK,),
        in_specs=[pl.BlockSpec((2, BB, AW), lambda b: (0, b, 0)),
                  _full((FH, 25)), _full((HF, HF)),
                  _full((HF, 16)), _full((HF, 16))],
        out_specs=_A_OUT_SPECS,
        out_shape=_A_OUTS,
    )(acc, rep, wbd, alp, arp)


def _tc_stage_b(src_raw, gmax):
    return pl.pallas_call(
        _tc_b_body,
        grid=(NBLK,),
        in_specs=[pl.BlockSpec((BB, 16), lambda b: (b, 0)),
                  _full((1, 16))],
        out_specs=pl.BlockSpec((2, BB, SW), lambda b: (0, b, 0)),
        out_shape=jax.ShapeDtypeStruct((2, NN, SW), jnp.float32),
    )(src_raw, gmax)


def _tc_final(acc, rep, wf, bf):
    return pl.pallas_call(
        _tc_final_body,
        grid=(NBLK,),
        in_specs=[pl.BlockSpec((2, BB, AW), lambda b: (0, b, 0)),
                  _full((FH, 25)), _full((HF, NC)), _full((1, NC))],
        out_specs=pl.BlockSpec((BB, NC), lambda b: (b, 0)),
        out_shape=jax.ShapeDtypeStruct((NN, NC), jnp.float32),
    )(acc, rep, wf, bf)


def _sc_edge_body(rows_hbm, cols_hbm, srct_hbm, bt_hbm, out_hbm,
                  ridx, cidx, srcb, bb, outb, zbuf, accum,
                  sem_i, sem_g, sem_s):
    c = lax.axis_index("c")
    s = lax.axis_index("s")
    zeros16 = jnp.zeros((16,), jnp.float32)
    i16 = lambda v: jnp.full((16,), v, jnp.int32)

    def _zb(i, carry):
        zbuf[i, pl.ds(0, 16)] = zeros16
        zbuf[i, pl.ds(16, 16)] = zeros16
        return carry
    lax.fori_loop(0, ZR, _zb, 0)

    base_row = s * RPT

    def _zc(i, carry):
        pltpu.sync_copy(zbuf, accum.at[pl.ds(base_row + i * ZR, ZR)])
        return carry
    lax.fori_loop(0, RPT // ZR, _zc, 0)

    def _zo(i, carry):
        e16 = lax.iota(jnp.int32, 16) + i * 16
        msk = e16 < 2 * CS
        plsc.store_scatter(outb, [e16, i16(30)], zeros16, mask=msk)
        plsc.store_scatter(outb, [e16, i16(31)], zeros16, mask=msk)
        return carry
    lax.fori_loop(0, (2 * CS + 15) // 16, _zo, 0)

    plsc.subcore_barrier()

    tile_r0 = s * (EPT // SUB)


    def _issue_idx(k, q):
        r0 = tile_r0 + k * 8
        pltpu.async_copy(rows_hbm.at[pl.ds(r0, 8)],
                         ridx.at[pl.ds(q * 8, 8)], sem_i)
        pltpu.async_copy(cols_hbm.at[pl.ds(r0, 8)],
                         cidx.at[pl.ds(q * 8, 8)], sem_i)

    def _drain_idx():
        pltpu.make_async_copy(rows_hbm.at[pl.ds(tile_r0, 8)],
                              ridx.at[pl.ds(0, 8)], sem_i).wait()
        pltpu.make_async_copy(cols_hbm.at[pl.ds(tile_r0, 8)],
                              cidx.at[pl.ds(0, 8)], sem_i).wait()

    def _rbp(g):
        k = g // 4
        rb = lax.rem(k, 2) * 8 + lax.rem(g, 4) * 2
        return rb, lax.rem(g, 2)

    def _issue_gathers(g):
        rb, p = _rbp(g)
        for j in range(2):
            pltpu.async_copy(srct_hbm.at[c].at[ridx.at[rb + j]],
                             srcb.at[pl.ds(p * CS + j * SUB, SUB)], sem_g)
            pltpu.async_copy(bt_hbm.at[c].at[cidx.at[rb + j]],
                             bb.at[pl.ds(p * CS + j * SUB, SUB)], sem_g)

    def _drain_gathers():
        for j in range(2):
            pltpu.make_async_copy(srct_hbm.at[c].at[ridx.at[j]],
                                  srcb.at[pl.ds(j * SUB, SUB)], sem_g).wait()
            pltpu.make_async_copy(bt_hbm.at[c].at[cidx.at[j]],
                                  bb.at[pl.ds(j * SUB, SUB)], sem_g).wait()

    def _issue_scatter(g):
        rb, p = _rbp(g)
        for j in range(2):
            pltpu.async_copy(outb.at[pl.ds(p * CS + j * SUB, SUB)],
                             accum.at[ridx.at[rb + j]], sem_s, add=True)

    def _drain_scatter():
        for j in range(2):
            pltpu.make_async_copy(outb.at[pl.ds(j * SUB, SUB)],
                                  accum.at[ridx.at[j]], sem_s).wait()

    def _grp_body(e16, msk):
        for h in range(5):
            hv = i16(h)
            sv = plsc.load_gather(srcb, [e16, hv], mask=msk)
            mv = plsc.load_gather(srcb, [e16, hv + 5], mask=msk)
            nv = plsc.load_gather(bb, [e16, hv], mask=msk)
            pre = sv + nv
            w = jnp.exp(jnp.maximum(pre, 0.2 * pre) - mv)
            plsc.store_scatter(outb, [e16, i16(25 + h)], w, mask=msk)
            for f in range(5):
                hp = plsc.load_gather(bb, [e16, i16(5 + h * 5 + f)], mask=msk)
                plsc.store_scatter(outb, [e16, i16(h * 5 + f)], hp * w,
                                   mask=msk)

    def _compute(g):
        base = lax.rem(g, 2) * CS

        def _grp(i, carry):
            _grp_body(lax.iota(jnp.int32, 16) + (base + i * 16), None)
            return carry
        lax.fori_loop(0, CS // 16, _grp, 0)
        tail = CS - (CS // 16) * 16
        if tail:
            e16 = lax.iota(jnp.int32, 16) + (base + CS - tail)
            _grp_body(e16, lax.iota(jnp.int32, 16) < tail)

    pltpu.sync_copy(rows_hbm.at[pl.ds(tile_r0, 8)], ridx.at[pl.ds(0, 8)])
    pltpu.sync_copy(cols_hbm.at[pl.ds(tile_r0, 8)], cidx.at[pl.ds(0, 8)])
    _issue_gathers(0)

    def _pipe(g, carry):
        t = lax.rem(g, 4)
        _drain_gathers()

        @pl.when(jnp.logical_and(t == 3, g // 4 + 1 < NIC))
        def _():
            _drain_idx()

        @pl.when(g + 1 < NSC)
        def _():
            _issue_gathers(g + 1)

        @pl.when(g >= 2)
        def _():
            _drain_scatter()

        @pl.when(t == 1)
        def _():
            k1 = g // 4 + 1

            @pl.when(k1 < NIC)
            def _():
                _issue_idx(k1, lax.rem(k1, 2))

        _compute(g)
        _issue_scatter(g)
        return carry
    lax.fori_loop(0, NSC, _pipe, 0)
    _drain_scatter()
    _drain_scatter()

    plsc.subcore_barrier()

    def _out(i, carry):
        rr = pl.ds(base_row + i * ZR, ZR)
        pltpu.sync_copy(accum.at[rr], out_hbm.at[c].at[rr])
        return carry
    lax.fori_loop(0, RPT // ZR, _out, 0)


def _sc_edge(rows2d, cols2d, srct, bt):
    mesh = plsc.VectorSubcoreMesh(core_axis_name="c", subcore_axis_name="s")
    f = pl.kernel(
        _sc_edge_body,
        out_type=jax.ShapeDtypeStruct((2, NP, AW), jnp.float32),
        mesh=mesh,
        compiler_params=pltpu.CompilerParams(needs_layout_passes=False,
                                             use_tc_tiling_on_sc=False),
        scratch_types=[
            pltpu.VMEM((16, SUB), jnp.int32),
            pltpu.VMEM((16, SUB), jnp.int32),
            pltpu.VMEM((2 * CS, SW), jnp.float32),
            pltpu.VMEM((2 * CS, AW), jnp.float32),
            pltpu.VMEM((2 * CS, AW), jnp.float32),
            pltpu.VMEM((ZR, AW), jnp.float32),
            pltpu.VMEM_SHARED((NP, AW), jnp.float32),
            pltpu.SemaphoreType.DMA,
            pltpu.SemaphoreType.DMA,
            pltpu.SemaphoreType.DMA,
        ],
    )
    return f(rows2d, cols2d, srct, bt)[:, :NN, :]


def kernel(x, edge_index, W_init, a_left, a_right, Ws, W_final, b_final):
    f32 = jnp.float32
    rows2d = edge_index[0].reshape(NE // SUB, SUB)
    cols2d = edge_index[1].reshape(NE // SUB, SUB)

    eye = jnp.eye(NH, dtype=f32)
    wbd = jnp.einsum('lhfo,hk->lhfko', Ws, eye).reshape(NL, HF, HF)
    alp = jnp.einsum('lhf,hk->lhfk', a_left, eye).reshape(NL, HF, NH)
    arp = jnp.einsum('lhf,hk->lhfk', a_right, eye).reshape(NL, HF, NH)
    zpad = jnp.zeros((NL, HF, 16 - NH), f32)
    alp = jnp.concatenate([alp, zpad], axis=2)
    arp = jnp.concatenate([arp, zpad], axis=2)
    rep = jnp.kron(jnp.eye(FH, dtype=f32), jnp.ones((1, FH), f32))

    bt, src_raw, gmax = _tc_stage_a0(x, W_init, wbd[0], alp[0], arp[0])
    srct = _tc_stage_b(src_raw, gmax)
    acc = _sc_edge(rows2d, cols2d, srct, bt)
    for i in range(1, NL):
        bt, src_raw, gmax = _tc_stage_ai(acc, rep, wbd[i], alp[i], arp[i])
        srct = _tc_stage_b(src_raw, gmax)
        acc = _sc_edge(rows2d, cols2d, srct, bt)
    return _tc_final(acc, rep, W_final, b_final.reshape(1, NC))
